# Initial kernel scaffold; baseline (speedup 1.0000x reference)
import jax
import jax.numpy as jnp
from jax.experimental import pallas as pl


def kernel(x, emb_tables, lin_tables, bias):
    raise NotImplementedError("write your pallas kernel here")


# speedup vs baseline: 1.5920x; 1.5920x over previous
import functools

import jax
import jax.numpy as jnp
from jax import lax
from jax.experimental import pallas as pl
from jax.experimental.pallas import tpu as pltpu
from jax.experimental.pallas import tpu_sc as plsc

F = 26
V = 100000
D = 32
B = 16384
NC = 2
NS = 16
NW = NC * NS
RPW = B // NW
CH = 128
NCH = RPW // CH
IPC = CH * F
NSTR = IPC // 128


def _fm_body(idx_hbm, emb_hbm, lin_hbm, bias_hbm, out_hbm,
             idx_v, rows_v, lin_v, out_v, bias_v, sem_e, sem_l):
    wid = lax.axis_index("s") * NC + lax.axis_index("c")
    base_row = wid * RPW

    pltpu.sync_copy(bias_hbm, bias_v.at[pl.ds(0, 1)])
    bias_s = bias_v[...][0]

    lane = lax.iota(jnp.int32, 16)
    lane_f = lane * F
    zero16 = jnp.zeros((16,), jnp.float32)

    def chunk_body(c, carry):
        row0 = base_row + c * CH
        i0 = row0 * F
        pltpu.sync_copy(idx_hbm.at[pl.ds(i0, IPC)], idx_v)
        cps = []
        for j in range(NSTR):
            sl = pl.ds(j * 128, 128)
            cps.append(pltpu.async_copy(emb_hbm.at[idx_v.at[sl]],
                                        rows_v.at[sl], sem_e))
            cps.append(pltpu.async_copy(lin_hbm.at[idx_v.at[sl]],
                                        lin_v.at[sl], sem_l))
        for cp in cps:
            cp.wait()

        def group_body(g, carry2):
            pos0 = g * (16 * F)
            lin_acc = zero16
            for f in range(F):
                lin_acc = lin_acc + plsc.load_gather(lin_v, [lane_f + (pos0 + f)])
            zacc = zero16
            for rr in range(16):
                r0 = pos0 + rr * F
                sa = zero16
                sb = zero16
                qa = zero16
                qb = zero16
                for f in range(F):
                    a = rows_v[r0 + f, pl.ds(0, 16)]
                    b = rows_v[r0 + f, pl.ds(16, 16)]
                    sa = sa + a
                    sb = sb + b
                    qa = qa + a * a
                    qb = qb + b * b
                p = sa * sa + sb * sb - qa - qb
                zacc = jnp.where(lane == rr, jnp.sum(p), zacc)
            z = zacc * 0.5 + lin_acc + bias_s
            out_v[pl.ds(g * 16, 16)] = 1.0 / (1.0 + jnp.exp(-z))
            return carry2

        lax.fori_loop(0, CH // 16, group_body, 0)
        pltpu.sync_copy(out_v, out_hbm.at[pl.ds(row0, CH)])
        return carry

    lax.fori_loop(0, NCH, chunk_body, 0)


@jax.jit
def _fm_call(flat_idx, emb2, lin1, bias):
    mesh = plsc.VectorSubcoreMesh(core_axis_name="c", subcore_axis_name="s")
    kern = pl.kernel(
        _fm_body,
        out_type=jax.ShapeDtypeStruct((B,), jnp.float32),
        mesh=mesh,
        scratch_types=[
            pltpu.VMEM((IPC,), jnp.int32),
            pltpu.VMEM((IPC, D), jnp.float32),
            pltpu.VMEM((IPC,), jnp.float32),
            pltpu.VMEM((CH,), jnp.float32),
            pltpu.VMEM((16,), jnp.float32),
            pltpu.SemaphoreType.DMA,
            pltpu.SemaphoreType.DMA,
        ],
        compiler_params=pltpu.CompilerParams(
            needs_layout_passes=False, use_tc_tiling_on_sc=False),
    )
    return kern(flat_idx, emb2, lin1, bias)


def kernel(x, emb_tables, lin_tables, bias):
    emb2 = emb_tables.reshape(F * V, D)
    lin1 = lin_tables.reshape(F * V)
    flat_idx = (x + jnp.arange(F, dtype=jnp.int32)[None, :] * V).reshape(-1)
    out = _fm_call(flat_idx, emb2, lin1, bias)
    return out.reshape(B, 1)


# speedup vs baseline: 1.5994x; 1.0046x over previous
import functools

import jax
import jax.numpy as jnp
from jax import lax
from jax.experimental import pallas as pl
from jax.experimental.pallas import tpu as pltpu
from jax.experimental.pallas import tpu_sc as plsc

F = 26
V = 100000
D = 32
B = 16384
NC = 2
NS = 16
NW = NC * NS
RPW = B // NW
CH = 128
NCH = RPW // CH
IPC = CH * F
NSTR = IPC // 128


def _fm_body(idx_hbm, emb_hbm, lin_hbm, bias_hbm, out_hbm,
             idx_v, rows_v, lin_v, out_v, bias_v, sem_e, sem_l):
    wid = lax.axis_index("s") * NC + lax.axis_index("c")
    base_row = wid * RPW

    pltpu.sync_copy(bias_hbm, bias_v.at[pl.ds(0, 1)])
    bias_s = bias_v[...][0]

    lane = lax.iota(jnp.int32, 16)
    lane_f = lane * F
    zero16 = jnp.zeros((16,), jnp.float32)

    def chunk_body(c, carry):
        row0 = base_row + c * CH
        i0 = row0 * F
        pltpu.sync_copy(idx_hbm.at[pl.ds(i0, IPC)], idx_v)
        cps = []
        for j in range(NSTR):
            sl = pl.ds(j * 128, 128)
            cps.append(pltpu.async_copy(emb_hbm.at[idx_v.at[sl]],
                                        rows_v.at[sl], sem_e))
            cps.append(pltpu.async_copy(lin_hbm.at[idx_v.at[sl]],
                                        lin_v.at[sl], sem_l))
        for cp in cps:
            cp.wait()

        def group_body(g, carry2):
            pos0 = g * (16 * F)
            lin_acc = zero16
            for f in range(F):
                lin_acc = lin_acc + plsc.load_gather(lin_v, [lane_f + (pos0 + f)])
            zacc = zero16
            for rr in range(16):
                r0 = pos0 + rr * F
                sa = zero16
                sb = zero16
                qa = zero16
                qb = zero16
                for f in range(F):
                    a = rows_v[r0 + f, pl.ds(0, 16)]
                    b = rows_v[r0 + f, pl.ds(16, 16)]
                    sa = sa + a
                    sb = sb + b
                    qa = qa + a * a
                    qb = qb + b * b
                p = sa * sa + sb * sb - qa - qb
                zacc = jnp.where(lane == rr, jnp.sum(p), zacc)
            z = zacc * 0.5 + lin_acc + bias_s
            out_v[pl.ds(g * 16, 16)] = 1.0 / (1.0 + jnp.exp(-z))
            return carry2

        lax.fori_loop(0, CH // 16, group_body, 0)
        pltpu.sync_copy(out_v, out_hbm.at[pl.ds(row0, CH)])
        return carry

    lax.fori_loop(0, NCH, chunk_body, 0)


@jax.jit
def _fm_call(flat_idx, emb2, lin1, bias):
    mesh = plsc.VectorSubcoreMesh(core_axis_name="c", subcore_axis_name="s")
    kern = pl.kernel(
        _fm_body,
        out_type=jax.ShapeDtypeStruct((B,), jnp.float32),
        mesh=mesh,
        scratch_types=[
            pltpu.VMEM((IPC,), jnp.int32),
            pltpu.VMEM((IPC, D), jnp.float32),
            pltpu.VMEM((IPC,), jnp.float32),
            pltpu.VMEM((CH,), jnp.float32),
            pltpu.VMEM((16,), jnp.float32),
            pltpu.SemaphoreType.DMA,
            pltpu.SemaphoreType.DMA,
        ],
        compiler_params=pltpu.CompilerParams(
            needs_layout_passes=False, use_tc_tiling_on_sc=False),
    )
    return kern(flat_idx, emb2, lin1, bias)


def kernel(x, emb_tables, lin_tables, bias):
    emb2 = emb_tables.reshape(F * V, D)
    lin1 = jnp.concatenate([lin_tables[i, :, 0] for i in range(F)], axis=0)
    flat_idx = (x + jnp.arange(F, dtype=jnp.int32)[None, :] * V).reshape(-1)
    out = _fm_call(flat_idx, emb2, lin1, bias)
    return out.reshape(B, 1)
